# baseline (speedup 1.0000x reference)
import jax
import jax.numpy as jnp
from jax import lax
from jax.experimental import pallas as pl
from jax.experimental.pallas import tpu as pltpu
from jax.experimental.pallas import tpu_sc as plsc

ND = 10000
NPAD = 10112
D = 128
NE = 320000
PADE = 327680
NSUB = 16
NCORE = 2
CHUNK = 128
SEG_CHUNKS = PADE // NSUB // CHUNK
DEC_CHUNKS = PADE // (NSUB * NCORE) // CHUNK
ROWS_PER_SUB = NPAD // NSUB

_F32 = jnp.float32
_I32 = jnp.int32


def _mesh():
    return plsc.VectorSubcoreMesh(core_axis_name="c", subcore_axis_name="s",
                                  num_cores=NCORE, num_subcores=NSUB)


IDX_GRP = 32
SEG_GROUPS = SEG_CHUNKS // IDX_GRP


def _segsum_body(s0a_hbm, s0b_hbm, s1a_hbm, s1b_hbm, row_hbm, col_hbm,
                 ag0_hbm, ag1_hbm,
                 rb, cb, gb0, gb1, acc, gsem0, gsem1, ssem0, ssem1):
    c = lax.axis_index("c")
    s = lax.axis_index("s")

    def _zrow(r, _):
        for k in range(8):
            gb0[r, pl.ds(k * 16, 16)] = jnp.zeros((16,), _F32)
        return 0
    lax.fori_loop(0, CHUNK, _zrow, 0)
    base = s * ROWS_PER_SUB
    for k in range(4):
        pltpu.sync_copy(gb0, acc.at[pl.ds(base + k * CHUNK, CHUNK)])
    pltpu.sync_copy(gb0.at[pl.ds(0, ROWS_PER_SUB - 4 * CHUNK)],
                    acc.at[pl.ds(base + 4 * CHUNK, ROWS_PER_SUB - 4 * CHUNK)])
    plsc.subcore_barrier()

    def _edge_loop(ct, srcA, srcB, srcC, srcD):
        def _group(g, _):
            pltpu.sync_copy(row_hbm.at[ct, s, pl.ds(g * IDX_GRP, IDX_GRP)], rb)
            pltpu.sync_copy(col_hbm.at[ct, s, pl.ds(g * IDX_GRP, IDX_GRP)], cb)
            pltpu.async_copy(srcA.at[rb.at[0]], gb0, gsem0)
            pltpu.async_copy(srcB.at[rb.at[1]], gb1, gsem1)

            def _step(i, buf, gsem, ssem, src, src_next, last):
                pltpu.make_async_copy(src.at[rb.at[i]], buf, gsem).wait()
                pltpu.async_copy(buf, acc.at[cb.at[i]], ssem, add=True)

                @pl.when(last)
                def _():
                    pltpu.make_async_copy(buf, acc.at[cb.at[i]], ssem).wait()
                    pltpu.async_copy(src_next.at[rb.at[i + 2]], buf, gsem)

            def _body(j, _):
                i0 = 4 * j
                more = j < IDX_GRP // 4 - 1
                _step(i0, gb0, gsem0, ssem0, srcA, srcC, True)
                _step(i0 + 1, gb1, gsem1, ssem1, srcB, srcD, True)
                _step(i0 + 2, gb0, gsem0, ssem0, srcC, srcA, more)
                _step(i0 + 3, gb1, gsem1, ssem1, srcD, srcB, more)
                return 0
            lax.fori_loop(0, IDX_GRP // 4, _body, 0)
            pltpu.make_async_copy(gb0, acc.at[cb.at[IDX_GRP - 2]], ssem0).wait()
            pltpu.make_async_copy(gb1, acc.at[cb.at[IDX_GRP - 1]], ssem1).wait()
            return 0
        lax.fori_loop(0, SEG_GROUPS, _group, 0)

    @pl.when(c == 0)
    def _():
        _edge_loop(0, s0a_hbm, s0b_hbm, s0a_hbm, s0b_hbm)

    @pl.when(c == 1)
    def _():
        _edge_loop(1, s1a_hbm, s1b_hbm, s1a_hbm, s1b_hbm)

    plsc.subcore_barrier()

    @pl.when(c == 0)
    def _():
        pltpu.sync_copy(acc.at[pl.ds(base, ROWS_PER_SUB)],
                        ag0_hbm.at[pl.ds(base, ROWS_PER_SUB)])

    @pl.when(c == 1)
    def _():
        pltpu.sync_copy(acc.at[pl.ds(base, ROWS_PER_SUB)],
                        ag1_hbm.at[pl.ds(base, ROWS_PER_SUB)])


def _segsum(s0a, s0b, s1a, s1b, row, col):
    return pl.kernel(
        _segsum_body,
        out_type=[jax.ShapeDtypeStruct((NPAD, D), _F32)] * 2,
        mesh=_mesh(),
        scratch_types=[
            pltpu.VMEM((IDX_GRP, CHUNK), _I32),
            pltpu.VMEM((IDX_GRP, CHUNK), _I32),
            pltpu.VMEM((CHUNK, D), _F32),
            pltpu.VMEM((CHUNK, D), _F32),
            pltpu.VMEM_SHARED((NPAD, D), _F32),
            pltpu.SemaphoreType.DMA,
            pltpu.SemaphoreType.DMA,
            pltpu.SemaphoreType.DMA,
            pltpu.SemaphoreType.DMA,
        ],
    )(s0a, s0b, s1a, s1b, row, col)


def _decoder_body(u_hbm, v_hbm, u2_hbm, v2_hbm, eli_hbm, w2_hbm, out_hbm,
                  row_v, col_v, ub0, vb0, ub1, vb1, w2_v, prow, sem0, sem1):
    c = lax.axis_index("c")
    s = lax.axis_index("s")
    wid = s * NCORE + c

    pltpu.sync_copy(eli_hbm.at[0, wid], row_v)
    pltpu.sync_copy(eli_hbm.at[1, wid], col_v)
    pltpu.sync_copy(w2_hbm, w2_v)
    w2r = [w2_v[pl.ds(k * 16, 16)] for k in range(8)]

    def _fire(i, ub, vb, sem, uh, vh):
        pltpu.async_copy(uh.at[row_v.at[i]], ub, sem)
        pltpu.async_copy(vh.at[col_v.at[i]], vb, sem)

    def _drain(i, ub, vb, sem, uh, vh):
        pltpu.make_async_copy(uh.at[row_v.at[i]], ub, sem).wait()
        pltpu.make_async_copy(vh.at[col_v.at[i]], vb, sem).wait()

    def _compute(i, ub, vb):
        def _ebody(e, _):
            acc = None
            for k in range(8):
                u = ub[e, pl.ds(k * 16, 16)]
                v = vb[e, pl.ds(k * 16, 16)]
                t = jnp.maximum(u + v, 0.0) * w2r[k]
                acc = t if acc is None else acc + t
            prow[e] = acc
            return 0
        lax.fori_loop(0, CHUNK, _ebody, 0)
        pltpu.sync_copy(prow, out_hbm.at[wid, i])

    _fire(0, ub0, vb0, sem0, u_hbm, v_hbm)

    def _body(j, _):
        i0 = 2 * j
        _fire(i0 + 1, ub1, vb1, sem1, u2_hbm, v2_hbm)
        _drain(i0, ub0, vb0, sem0, u_hbm, v_hbm)
        _compute(i0, ub0, vb0)

        @pl.when(j < DEC_CHUNKS // 2 - 1)
        def _():
            _fire(i0 + 2, ub0, vb0, sem0, u_hbm, v_hbm)
        _drain(i0 + 1, ub1, vb1, sem1, u2_hbm, v2_hbm)
        _compute(i0 + 1, ub1, vb1)
        return 0
    lax.fori_loop(0, DEC_CHUNKS // 2, _body, 0)


def _decoder(u, v, u2, v2, eli, w2cat):
    return pl.kernel(
        _decoder_body,
        out_type=jax.ShapeDtypeStruct((NSUB * NCORE, DEC_CHUNKS, CHUNK, 16), _F32),
        mesh=_mesh(),
        scratch_types=[
            pltpu.VMEM((DEC_CHUNKS, CHUNK), _I32),
            pltpu.VMEM((DEC_CHUNKS, CHUNK), _I32),
            pltpu.VMEM((CHUNK, D), _F32),
            pltpu.VMEM((CHUNK, D), _F32),
            pltpu.VMEM((CHUNK, D), _F32),
            pltpu.VMEM((CHUNK, D), _F32),
            pltpu.VMEM((D + 16, ), _F32),
            pltpu.VMEM((CHUNK, 16), _F32),
            pltpu.SemaphoreType.DMA,
            pltpu.SemaphoreType.DMA,
        ],
    )(u, v, u2, v2, eli, w2cat)


def _psum_body(p_ref, m_ref, b2_ref, o_ref):
    o_ref[...] = (jnp.dot(p_ref[...], m_ref[...], preferred_element_type=_F32)
                  + b2_ref[...])


def _psum(p2, m, b2row):
    n = PADE * 16 // D
    blk = n // 8
    return pl.pallas_call(
        _psum_body,
        out_shape=jax.ShapeDtypeStruct((n, 8), _F32),
        grid=(8,),
        in_specs=[
            pl.BlockSpec((blk, D), lambda i: (i, 0)),
            pl.BlockSpec((D, 8), lambda i: (0, 0)),
            pl.BlockSpec((1, 8), lambda i: (0, 0)),
        ],
        out_specs=pl.BlockSpec((blk, 8), lambda i: (i, 0)),
    )(p2, m, b2row)


def _tc_layer_body(ag0, ag1, s0, s1, wlpd, blpd, wrpd,
                   wldp, bldp, wrdp, o0, o1, o0b, o1b):
    nd = (jnp.dot(ag0[...], wlpd[...], preferred_element_type=_F32)
          + jnp.dot(s1[...], wrpd[...], preferred_element_type=_F32)
          + blpd[...])
    np_ = (jnp.dot(ag1[...], wldp[...], preferred_element_type=_F32)
           + jnp.dot(s0[...], wrdp[...], preferred_element_type=_F32)
           + bldp[...])
    nd, np_ = jnp.tanh(nd), jnp.tanh(np_)
    o1[...] = nd
    o0[...] = np_
    o1b[...] = nd
    o0b[...] = np_


def _tc_layer(ag0, ag1, s0, s1, wlpd, blpd, wrpd, wldp, bldp, wrdp):
    return pl.pallas_call(
        _tc_layer_body,
        out_shape=[jax.ShapeDtypeStruct((NPAD, D), _F32)] * 4,
    )(ag0, ag1, s0, s1, wlpd, blpd, wrpd, wldp, bldp, wrdp)


def _tc_final_body(ag0, ag1, s0, s1, wlpd, blpd, wrpd, wldp, bldp, wrdp,
                   w1a, w1b, b1, o0, o1, u, v, ub, vb):
    nd = (jnp.dot(ag0[...], wlpd[...], preferred_element_type=_F32)
          + jnp.dot(s1[...], wrpd[...], preferred_element_type=_F32)
          + blpd[...])
    np_ = (jnp.dot(ag1[...], wldp[...], preferred_element_type=_F32)
           + jnp.dot(s0[...], wrdp[...], preferred_element_type=_F32)
           + bldp[...])
    o1[...] = nd
    o0[...] = np_
    uu = jnp.dot(nd, w1a[...], preferred_element_type=_F32) + b1[...]
    vv = jnp.dot(np_, w1b[...], preferred_element_type=_F32)
    u[...] = uu
    v[...] = vv
    ub[...] = uu
    vb[...] = vv


def _tc_final(ag0, ag1, s0, s1, wlpd, blpd, wrpd, wldp, bldp, wrdp, w1a, w1b, b1):
    return pl.pallas_call(
        _tc_final_body,
        out_shape=[jax.ShapeDtypeStruct((NPAD, D), _F32)] * 6,
    )(ag0, ag1, s0, s1, wlpd, blpd, wrpd, wldp, bldp, wrdp, w1a, w1b, b1)


def _dup_body(a, b, oa, ob):
    oa[...] = a[...]
    ob[...] = b[...]


def _dup(a, b):
    return pl.pallas_call(
        _dup_body,
        out_shape=[jax.ShapeDtypeStruct((NPAD, D), _F32)] * 2,
    )(a, b)


def _pad_nodes(x):
    return jnp.concatenate([x, jnp.zeros((NPAD - ND, D), _F32)], axis=0)


def _prep_seg_edges(ei):
    row = jnp.concatenate([ei[0], jnp.zeros((PADE - NE,), _I32)])
    col = jnp.concatenate([ei[1], jnp.full((PADE - NE,), ND, _I32)])
    return (row.reshape(NSUB, SEG_CHUNKS, CHUNK),
            col.reshape(NSUB, SEG_CHUNKS, CHUNK))


def kernel(x_drug, x_protein, params, edge_index_drug_to_protein,
           edge_index_protein_to_drug, edge_label_index):
    p = params
    xd = _pad_nodes(x_drug)
    xp = _pad_nodes(x_protein)

    row_pd, col_pd = _prep_seg_edges(edge_index_protein_to_drug)
    row_dp, col_dp = _prep_seg_edges(edge_index_drug_to_protein)
    ROW = jnp.stack([row_pd, row_dp])
    COL = jnp.stack([col_pd, col_dp])

    eli0 = jnp.concatenate([edge_label_index[0], jnp.zeros((PADE - NE,), _I32)])
    eli1 = jnp.concatenate([edge_label_index[1], jnp.zeros((PADE - NE,), _I32)])
    ELI = jnp.stack([eli0.reshape(NSUB * NCORE, DEC_CHUNKS, CHUNK),
                     eli1.reshape(NSUB * NCORE, DEC_CHUNKS, CHUNK)])

    w2cat = jnp.concatenate([p['W2'].reshape(-1), jnp.zeros((16,), _F32)])
    m_sum = jnp.repeat(jnp.eye(8, dtype=_F32), 16, axis=0)
    b2row = jnp.broadcast_to(p['b2'].reshape(1, 1), (1, 8)).astype(_F32)
    w1a = p['W1'][:D]
    w1b = p['W1'][D:]

    def wts(name):
        return (p['Wl_%s_pd' % name], p['bl_%s_pd' % name].reshape(1, D),
                p['Wr_%s_pd' % name], p['Wl_%s_dp' % name],
                p['bl_%s_dp' % name].reshape(1, D), p['Wr_%s_dp' % name])

    s0, s1 = xp, xd
    s0b, s1b = _dup(xp, xd)
    for name in ('in', 'med', 'med'):
        ag0, ag1 = _segsum(s0, s0b, s1, s1b, ROW, COL)
        s0, s1, s0b, s1b = _tc_layer(ag0, ag1, s0, s1, *wts(name))
    ag0, ag1 = _segsum(s0, s0b, s1, s1b, ROW, COL)
    s0, s1, U, V, Ub, Vb = _tc_final(ag0, ag1, s0, s1, *wts('out'), w1a, w1b,
                                     p['b1'].reshape(1, D))

    pparts = _decoder(U, V, Ub, Vb, ELI, w2cat)
    s8 = _psum(pparts.reshape(-1, D), m_sum, b2row)
    return (s1[:ND], s0[:ND], s8.reshape(-1)[:NE])


# speedup vs baseline: 1.0029x; 1.0029x over previous
import jax
import jax.numpy as jnp
from jax import lax
from jax.experimental import pallas as pl
from jax.experimental.pallas import tpu as pltpu
from jax.experimental.pallas import tpu_sc as plsc

ND = 10000
NPAD = 10112
D = 128
NE = 320000
PADE = 327680
NSUB = 16
NCORE = 2
CHUNK = 128
SEG_CHUNKS = PADE // NSUB // CHUNK
DEC_CHUNKS = PADE // (NSUB * NCORE) // CHUNK
ROWS_PER_SUB = NPAD // NSUB

_F32 = jnp.float32
_I32 = jnp.int32


def _mesh():
    return plsc.VectorSubcoreMesh(core_axis_name="c", subcore_axis_name="s",
                                  num_cores=NCORE, num_subcores=NSUB)


IDX_GRP = 32
SEG_GROUPS = SEG_CHUNKS // IDX_GRP


def _segsum_body(s0a_hbm, s0b_hbm, s1a_hbm, s1b_hbm, row_hbm, col_hbm,
                 ag0_hbm, ag1_hbm,
                 rb, cb, gb0, gb1, acc, gsem0, gsem1, ssem0, ssem1):
    c = lax.axis_index("c")
    s = lax.axis_index("s")

    def _zrow(r, _):
        for k in range(8):
            gb0[r, pl.ds(k * 16, 16)] = jnp.zeros((16,), _F32)
        return 0
    lax.fori_loop(0, CHUNK, _zrow, 0)
    base = s * ROWS_PER_SUB
    for k in range(4):
        pltpu.sync_copy(gb0, acc.at[pl.ds(base + k * CHUNK, CHUNK)])
    pltpu.sync_copy(gb0.at[pl.ds(0, ROWS_PER_SUB - 4 * CHUNK)],
                    acc.at[pl.ds(base + 4 * CHUNK, ROWS_PER_SUB - 4 * CHUNK)])
    plsc.subcore_barrier()

    def _edge_loop(ct, srcA, srcB, srcC, srcD):
        def _group(g, _):
            pltpu.sync_copy(row_hbm.at[ct, s, pl.ds(g * IDX_GRP, IDX_GRP)], rb)
            pltpu.sync_copy(col_hbm.at[ct, s, pl.ds(g * IDX_GRP, IDX_GRP)], cb)
            pltpu.async_copy(srcA.at[rb.at[0]], gb0, gsem0)
            pltpu.async_copy(srcB.at[rb.at[1]], gb1, gsem1)

            def _step(i, buf, gsem, ssem, src, src_next, last):
                pltpu.make_async_copy(src.at[rb.at[i]], buf, gsem).wait()
                pltpu.async_copy(buf, acc.at[cb.at[i]], ssem, add=True)

                @pl.when(last)
                def _():
                    pltpu.make_async_copy(buf, acc.at[cb.at[i]], ssem).wait()
                    pltpu.async_copy(src_next.at[rb.at[i + 2]], buf, gsem)

            def _body(j, _):
                i0 = 4 * j
                more = j < IDX_GRP // 4 - 1
                _step(i0, gb0, gsem0, ssem0, srcA, srcC, True)
                _step(i0 + 1, gb1, gsem1, ssem1, srcB, srcD, True)
                _step(i0 + 2, gb0, gsem0, ssem0, srcC, srcA, more)
                _step(i0 + 3, gb1, gsem1, ssem1, srcD, srcB, more)
                return 0
            lax.fori_loop(0, IDX_GRP // 4, _body, 0)
            pltpu.make_async_copy(gb0, acc.at[cb.at[IDX_GRP - 2]], ssem0).wait()
            pltpu.make_async_copy(gb1, acc.at[cb.at[IDX_GRP - 1]], ssem1).wait()
            return 0
        lax.fori_loop(0, SEG_GROUPS, _group, 0)

    @pl.when(c == 0)
    def _():
        _edge_loop(0, s0a_hbm, s0b_hbm, s0a_hbm, s0b_hbm)

    @pl.when(c == 1)
    def _():
        _edge_loop(1, s1a_hbm, s1b_hbm, s1a_hbm, s1b_hbm)

    plsc.subcore_barrier()

    @pl.when(c == 0)
    def _():
        pltpu.sync_copy(acc.at[pl.ds(base, ROWS_PER_SUB)],
                        ag0_hbm.at[pl.ds(base, ROWS_PER_SUB)])

    @pl.when(c == 1)
    def _():
        pltpu.sync_copy(acc.at[pl.ds(base, ROWS_PER_SUB)],
                        ag1_hbm.at[pl.ds(base, ROWS_PER_SUB)])


def _segsum(s0a, s0b, s1a, s1b, row, col):
    return pl.kernel(
        _segsum_body,
        out_type=[jax.ShapeDtypeStruct((NPAD, D), _F32)] * 2,
        mesh=_mesh(),
        scratch_types=[
            pltpu.VMEM((IDX_GRP, CHUNK), _I32),
            pltpu.VMEM((IDX_GRP, CHUNK), _I32),
            pltpu.VMEM((CHUNK, D), _F32),
            pltpu.VMEM((CHUNK, D), _F32),
            pltpu.VMEM_SHARED((NPAD, D), _F32),
            pltpu.SemaphoreType.DMA,
            pltpu.SemaphoreType.DMA,
            pltpu.SemaphoreType.DMA,
            pltpu.SemaphoreType.DMA,
        ],
    )(s0a, s0b, s1a, s1b, row, col)


def _decoder_body(u_hbm, v_hbm, u2_hbm, v2_hbm, eli_hbm, w2_hbm, out_hbm,
                  row_v, col_v, ub0, vb0, ub1, vb1, w2_v, prow, sem0, sem1):
    c = lax.axis_index("c")
    s = lax.axis_index("s")
    wid = s * NCORE + c

    pltpu.sync_copy(eli_hbm.at[0, wid], row_v)
    pltpu.sync_copy(eli_hbm.at[1, wid], col_v)
    pltpu.sync_copy(w2_hbm, w2_v)
    w2r = [w2_v[pl.ds(k * 16, 16)] for k in range(8)]

    def _fire(i, ub, vb, sem, uh, vh):
        pltpu.async_copy(uh.at[row_v.at[i]], ub, sem)
        pltpu.async_copy(vh.at[col_v.at[i]], vb, sem)

    def _drain(i, ub, vb, sem, uh, vh):
        pltpu.make_async_copy(uh.at[row_v.at[i]], ub, sem).wait()
        pltpu.make_async_copy(vh.at[col_v.at[i]], vb, sem).wait()

    def _compute(i, ub, vb):
        def _ebody(e, _):
            acc = None
            for k in range(8):
                u = ub[e, pl.ds(k * 16, 16)]
                v = vb[e, pl.ds(k * 16, 16)]
                t = jnp.maximum(u + v, 0.0) * w2r[k]
                acc = t if acc is None else acc + t
            prow[e] = acc
            return 0
        lax.fori_loop(0, CHUNK, _ebody, 0)
        pltpu.sync_copy(prow, out_hbm.at[wid, i])

    _fire(0, ub0, vb0, sem0, u_hbm, v_hbm)

    def _body(j, _):
        i0 = 2 * j
        _fire(i0 + 1, ub1, vb1, sem1, u2_hbm, v2_hbm)
        _drain(i0, ub0, vb0, sem0, u_hbm, v_hbm)
        _compute(i0, ub0, vb0)

        @pl.when(j < DEC_CHUNKS // 2 - 1)
        def _():
            _fire(i0 + 2, ub0, vb0, sem0, u_hbm, v_hbm)
        _drain(i0 + 1, ub1, vb1, sem1, u2_hbm, v2_hbm)
        _compute(i0 + 1, ub1, vb1)
        return 0
    lax.fori_loop(0, DEC_CHUNKS // 2, _body, 0)


def _decoder(u, v, u2, v2, eli, w2cat):
    return pl.kernel(
        _decoder_body,
        out_type=jax.ShapeDtypeStruct((NSUB * NCORE, DEC_CHUNKS, CHUNK, 16), _F32),
        mesh=_mesh(),
        scratch_types=[
            pltpu.VMEM((DEC_CHUNKS, CHUNK), _I32),
            pltpu.VMEM((DEC_CHUNKS, CHUNK), _I32),
            pltpu.VMEM((CHUNK, D), _F32),
            pltpu.VMEM((CHUNK, D), _F32),
            pltpu.VMEM((CHUNK, D), _F32),
            pltpu.VMEM((CHUNK, D), _F32),
            pltpu.VMEM((D + 16, ), _F32),
            pltpu.VMEM((CHUNK, 16), _F32),
            pltpu.SemaphoreType.DMA,
            pltpu.SemaphoreType.DMA,
        ],
    )(u, v, u2, v2, eli, w2cat)


def _psum_body(p_ref, m_ref, b2_ref, o_ref):
    o_ref[...] = (jnp.dot(p_ref[...], m_ref[...], preferred_element_type=_F32)
                  + b2_ref[...])


def _psum(p2, m, b2row):
    n = PADE * 16 // D
    blk = n // 8
    return pl.pallas_call(
        _psum_body,
        out_shape=jax.ShapeDtypeStruct((n, 8), _F32),
        grid=(8,),
        in_specs=[
            pl.BlockSpec((blk, D), lambda i: (i, 0)),
            pl.BlockSpec((D, 8), lambda i: (0, 0)),
            pl.BlockSpec((1, 8), lambda i: (0, 0)),
        ],
        out_specs=pl.BlockSpec((blk, 8), lambda i: (i, 0)),
    )(p2, m, b2row)


def _tc_layer_body(ag0, ag1, s0, s1, wlpd, blpd, wrpd,
                   wldp, bldp, wrdp, o0, o1, o0b, o1b):
    nd = (jnp.dot(ag0[...], wlpd[...], preferred_element_type=_F32)
          + jnp.dot(s1[...], wrpd[...], preferred_element_type=_F32)
          + blpd[...])
    np_ = (jnp.dot(ag1[...], wldp[...], preferred_element_type=_F32)
           + jnp.dot(s0[...], wrdp[...], preferred_element_type=_F32)
           + bldp[...])
    nd, np_ = jnp.tanh(nd), jnp.tanh(np_)
    o1[...] = nd
    o0[...] = np_
    o1b[...] = nd
    o0b[...] = np_


_BR = NPAD // 4


def _row_spec():
    return pl.BlockSpec((_BR, D), lambda i: (i, 0))


def _w_spec():
    return pl.BlockSpec((D, D), lambda i: (0, 0))


def _b_spec():
    return pl.BlockSpec((1, D), lambda i: (0, 0))


def _tc_layer(ag0, ag1, s0, s1, wlpd, blpd, wrpd, wldp, bldp, wrdp):
    return pl.pallas_call(
        _tc_layer_body,
        grid=(4,),
        in_specs=[_row_spec()] * 4
        + [_w_spec(), _b_spec(), _w_spec(), _w_spec(), _b_spec(), _w_spec()],
        out_specs=[_row_spec()] * 4,
        out_shape=[jax.ShapeDtypeStruct((NPAD, D), _F32)] * 4,
    )(ag0, ag1, s0, s1, wlpd, blpd, wrpd, wldp, bldp, wrdp)


def _tc_final_body(ag0, ag1, s0, s1, wlpd, blpd, wrpd, wldp, bldp, wrdp,
                   w1a, w1b, b1, o0, o1, u, v, ub, vb):
    nd = (jnp.dot(ag0[...], wlpd[...], preferred_element_type=_F32)
          + jnp.dot(s1[...], wrpd[...], preferred_element_type=_F32)
          + blpd[...])
    np_ = (jnp.dot(ag1[...], wldp[...], preferred_element_type=_F32)
           + jnp.dot(s0[...], wrdp[...], preferred_element_type=_F32)
           + bldp[...])
    o1[...] = nd
    o0[...] = np_
    uu = jnp.dot(nd, w1a[...], preferred_element_type=_F32) + b1[...]
    vv = jnp.dot(np_, w1b[...], preferred_element_type=_F32)
    u[...] = uu
    v[...] = vv
    ub[...] = uu
    vb[...] = vv


def _tc_final(ag0, ag1, s0, s1, wlpd, blpd, wrpd, wldp, bldp, wrdp, w1a, w1b, b1):
    return pl.pallas_call(
        _tc_final_body,
        grid=(4,),
        in_specs=[_row_spec()] * 4
        + [_w_spec(), _b_spec(), _w_spec(), _w_spec(), _b_spec(), _w_spec(),
           _w_spec(), _w_spec(), _b_spec()],
        out_specs=[_row_spec()] * 6,
        out_shape=[jax.ShapeDtypeStruct((NPAD, D), _F32)] * 6,
    )(ag0, ag1, s0, s1, wlpd, blpd, wrpd, wldp, bldp, wrdp, w1a, w1b, b1)


def _dup_body(a, b, oa, ob):
    oa[...] = a[...]
    ob[...] = b[...]


def _dup(a, b):
    return pl.pallas_call(
        _dup_body,
        grid=(4,),
        in_specs=[_row_spec()] * 2,
        out_specs=[_row_spec()] * 2,
        out_shape=[jax.ShapeDtypeStruct((NPAD, D), _F32)] * 2,
    )(a, b)


def _pad_nodes(x):
    return jnp.concatenate([x, jnp.zeros((NPAD - ND, D), _F32)], axis=0)


def _prep_seg_edges(ei):
    row = jnp.concatenate([ei[0], jnp.zeros((PADE - NE,), _I32)])
    col = jnp.concatenate([ei[1], jnp.full((PADE - NE,), ND, _I32)])
    return (row.reshape(NSUB, SEG_CHUNKS, CHUNK),
            col.reshape(NSUB, SEG_CHUNKS, CHUNK))


def kernel(x_drug, x_protein, params, edge_index_drug_to_protein,
           edge_index_protein_to_drug, edge_label_index):
    p = params
    xd = _pad_nodes(x_drug)
    xp = _pad_nodes(x_protein)

    row_pd, col_pd = _prep_seg_edges(edge_index_protein_to_drug)
    row_dp, col_dp = _prep_seg_edges(edge_index_drug_to_protein)
    ROW = jnp.stack([row_pd, row_dp])
    COL = jnp.stack([col_pd, col_dp])

    eli0 = jnp.concatenate([edge_label_index[0], jnp.zeros((PADE - NE,), _I32)])
    eli1 = jnp.concatenate([edge_label_index[1], jnp.zeros((PADE - NE,), _I32)])
    ELI = jnp.stack([eli0.reshape(NSUB * NCORE, DEC_CHUNKS, CHUNK),
                     eli1.reshape(NSUB * NCORE, DEC_CHUNKS, CHUNK)])

    w2cat = jnp.concatenate([p['W2'].reshape(-1), jnp.zeros((16,), _F32)])
    m_sum = jnp.repeat(jnp.eye(8, dtype=_F32), 16, axis=0)
    b2row = jnp.broadcast_to(p['b2'].reshape(1, 1), (1, 8)).astype(_F32)
    w1a = p['W1'][:D]
    w1b = p['W1'][D:]

    def wts(name):
        return (p['Wl_%s_pd' % name], p['bl_%s_pd' % name].reshape(1, D),
                p['Wr_%s_pd' % name], p['Wl_%s_dp' % name],
                p['bl_%s_dp' % name].reshape(1, D), p['Wr_%s_dp' % name])

    s0, s1 = xp, xd
    s0b, s1b = _dup(xp, xd)
    for name in ('in', 'med', 'med'):
        ag0, ag1 = _segsum(s0, s0b, s1, s1b, ROW, COL)
        s0, s1, s0b, s1b = _tc_layer(ag0, ag1, s0, s1, *wts(name))
    ag0, ag1 = _segsum(s0, s0b, s1, s1b, ROW, COL)
    s0, s1, U, V, Ub, Vb = _tc_final(ag0, ag1, s0, s1, *wts('out'), w1a, w1b,
                                     p['b1'].reshape(1, D))

    pparts = _decoder(U, V, Ub, Vb, ELI, w2cat)
    s8 = _psum(pparts.reshape(-1, D), m_sum, b2row)
    return (s1[:ND], s0[:ND], s8.reshape(-1)[:NE])
